# Initial kernel scaffold; baseline (speedup 1.0000x reference)
import jax
import jax.numpy as jnp
from jax.experimental import pallas as pl


def kernel(xyz, feat, params):
    raise NotImplementedError("write your pallas kernel here")


# speedup vs baseline: 12.4908x; 12.4908x over previous
import functools

import jax
import jax.numpy as jnp
from jax import lax
from jax.experimental import pallas as pl
from jax.experimental.pallas import tpu as pltpu
from jax.experimental.pallas import tpu_sc as plsc

BATCH = 2
N = 1024
D = 128
KNN = 128
NLAYERS = 4
BLK = 128
NBLK = N // BLK
NCHUNK = N // 128

_f32 = jnp.float32
_i32 = jnp.int32


def _dgT(a, b):
    return lax.dot_general(a, b, (((1,), (1,)), ((), ())),
                           preferred_element_type=_f32)


def _prep_body(wk_ref, wv_ref, wp2_ref, wo_ref, bv_ref, bp2_ref, bo_ref,
               wvwo_ref, wp2wo_ref, misc_ref):
    wo = wo_ref[0]
    wvwo_ref[0] = jnp.dot(wv_ref[0], wo, preferred_element_type=_f32)
    wp2wo_ref[0] = jnp.dot(wp2_ref[0], wo, preferred_element_type=_f32)
    ones = jnp.ones((1, D), _f32)
    wks = _dgT(ones, wk_ref[0])
    w2s = _dgT(ones, wp2_ref[0])
    cvec = lax.dot_general(bv_ref[0] + bp2_ref[0], wo,
                           (((1,), (0,)), ((), ())),
                           preferred_element_type=_f32) + bo_ref[0]
    misc_ref[0] = jnp.concatenate(
        [wks, w2s, cvec, jnp.zeros((5, 128), _f32)], axis=0)


def _prep(params):
    wk = jnp.stack([p['Wk'] for p in params])
    wv = jnp.stack([p['Wv'] for p in params])
    wp2 = jnp.stack([p['Wp2'] for p in params])
    wo = jnp.stack([p['Wo'] for p in params])
    bv = jnp.stack([p['bv'][None, :] for p in params])
    bp2 = jnp.stack([p['bp2'][None, :] for p in params])
    bo = jnp.stack([p['bo'][None, :] for p in params])
    full = pl.BlockSpec((1, D, D), lambda l: (l, 0, 0))
    row = pl.BlockSpec((1, 1, D), lambda l: (l, 0, 0))
    return pl.pallas_call(
        _prep_body,
        grid=(NLAYERS,),
        in_specs=[full, full, full, full, row, row, row],
        out_specs=[full, full, pl.BlockSpec((1, 8, 128), lambda l: (l, 0, 0))],
        out_shape=[
            jax.ShapeDtypeStruct((NLAYERS, D, D), _f32),
            jax.ShapeDtypeStruct((NLAYERS, D, D), _f32),
            jax.ShapeDtypeStruct((NLAYERS, 8, 128), _f32),
        ],
    )(wk, wv, wp2, wo, bv, bp2, bo)


def _sortable(d):
    i = lax.bitcast_convert_type(d, _i32)
    return jnp.where(i >= 0, i, i ^ jnp.int32(0x7FFFFFFF))


def _lane_cumsum(x):
    acc = x
    sh = 1
    while sh < N:
        z = jnp.zeros((BLK, sh), _f32)
        acc = acc + jnp.concatenate([z, acc[:, :-sh]], axis=1)
        sh *= 2
    return acc


def _knn_body(xyz_ref, xyzb_ref, pos_ref, xyzT_ref):
    i = pl.program_id(1)
    xyzf = xyz_ref[0]
    blk = xyzb_ref[0]
    s2 = jnp.sum(blk * blk, axis=1, keepdims=True)
    eye3 = jnp.eye(3, dtype=_f32)
    key_chunks = []
    xt_rows = []
    for hi in range(NCHUNK):
        ch = xyzf[hi * 128:(hi + 1) * 128, :]
        cross = _dgT(blk, ch)
        d2 = _dgT(jnp.ones((1, 3), _f32), ch * ch)
        dist = s2 + d2 - 2.0 * cross
        key_chunks.append(_sortable(dist))
        xt_rows.append(_dgT(eye3, ch))
    keys = jnp.concatenate(key_chunks, axis=1)

    @pl.when(i == 0)
    def _():
        xyzT_ref[0] = jnp.concatenate(
            [jnp.concatenate([xt_rows[hi][c:c + 1, :] for hi in range(NCHUNK)],
                             axis=0) for c in range(3)], axis=0)

    ones_row = jnp.ones((1, N), _f32)
    kf = jnp.float32(KNN)
    lo = jnp.full((BLK, 1), jnp.iinfo(jnp.int32).min, _i32)
    hi_b = jnp.full((BLK, 1), jnp.iinfo(jnp.int32).max, _i32)
    for _ in range(32):
        mid = (lo >> 1) + (hi_b >> 1) + (lo & hi_b & 1)
        ind = (keys <= mid).astype(_f32)
        cnt = _dgT(ind, ones_row)
        pred = cnt >= kf
        hi_b = jnp.where(pred, mid, hi_b)
        lo = jnp.where(pred, lo, mid + 1)
    thr = lo

    sel_lt = (keys < thr).astype(_f32)
    ties = (keys == thr).astype(_f32)
    cnt_lt = _dgT(sel_lt, ones_row)
    need = kf - cnt_lt
    tie_excl = _lane_cumsum(ties) - ties
    sel = sel_lt + ties * (tie_excl < need).astype(_f32)
    rank_excl = _lane_cumsum(sel) - sel
    pos_ref[0] = jnp.where(sel > 0.5, rank_excl.astype(_i32), jnp.int32(-1))


def _knn(xyz):
    return pl.pallas_call(
        _knn_body,
        grid=(BATCH, NBLK),
        in_specs=[pl.BlockSpec((1, N, 3), lambda b, i: (b, 0, 0)),
                  pl.BlockSpec((1, BLK, 3), lambda b, i: (b, i, 0))],
        out_specs=[
            pl.BlockSpec((1, BLK, N), lambda b, i: (b, i, 0)),
            pl.BlockSpec((1, 24, 128), lambda b, i: (b, 0, 0)),
        ],
        out_shape=[
            jax.ShapeDtypeStruct((BATCH, N, N), _i32),
            jax.ShapeDtypeStruct((BATCH, 24, 128), _f32),
        ],
    )(xyz, xyz)


_ROWS = BATCH * N
_NW = 32
_RPW = _ROWS // _NW
_RB = 8


def _sc_compact_body(pos_hbm, idx_hbm, pos_v, idx_v):
    wid = lax.axis_index("s") * 2 + lax.axis_index("c")

    def outer(t, _):
        base = wid * _RPW + t * _RB
        pltpu.sync_copy(pos_hbm.at[pl.ds(base * N, _RB * N)], pos_v)

        def chunk_body(ci, _):
            pv = pos_v[pl.ds(ci * 16, 16)]
            mask = pv >= 0
            pvc = jnp.maximum(pv, 0)
            r = (ci * 16) // N
            jv = lax.iota(_i32, 16) + (ci * 16 - r * N)
            plsc.store_scatter(idx_v, [pvc + r * KNN], jv, mask=mask)
            return 0

        lax.fori_loop(0, _RB * N // 16, chunk_body, 0)
        pltpu.sync_copy(idx_v, idx_hbm.at[pl.ds(base * KNN, _RB * KNN)])
        return 0

    lax.fori_loop(0, _RPW // _RB, outer, 0)


def _sc_compact(pos):
    mesh = plsc.VectorSubcoreMesh(core_axis_name="c", subcore_axis_name="s")
    kern = functools.partial(
        pl.kernel,
        mesh=mesh,
        compiler_params=pltpu.CompilerParams(needs_layout_passes=False),
        out_type=jax.ShapeDtypeStruct((_ROWS * KNN,), _i32),
        scratch_types=[
            pltpu.VMEM((_RB * N,), _i32),
            pltpu.VMEM((_RB * KNN,), _i32),
        ],
    )(_sc_compact_body)
    idx = kern(pos.reshape(_ROWS * N))
    return idx.reshape(BATCH, N, KNN)


def _gather_lane(table_row, idx_lo):
    t = jnp.broadcast_to(table_row, (BLK, 128))
    return jnp.take_along_axis(t, idx_lo, axis=1)


def _layer_body(feat_ref, featb_ref, xyz_ref, xyzT_ref, idx_ref, pos_ref,
                wvwo_ref, wp2wo_ref, misc_ref, wp1_ref, bp1_ref, gamma_ref,
                out_ref):
    featf = feat_ref[0]
    vfo = jnp.dot(featf, wvwo_ref[...], preferred_element_type=_f32)
    wks_row = misc_ref[0:1, :]
    w2s_row = misc_ref[1:2, :]
    cvec = misc_ref[2:3, :]

    idxb = idx_ref[0]
    idx_hi = idxb >> 7
    idx_lo = idxb & 127

    xyzT = xyzT_ref[0]
    ksg = jnp.zeros((BLK, 128), _f32)
    nx = jnp.zeros((BLK, 128), _f32)
    ny = jnp.zeros((BLK, 128), _f32)
    nz = jnp.zeros((BLK, 128), _f32)
    for hi in range(NCHUNK):
        m = (idx_hi == hi)
        ks_row = _dgT(wks_row, featf[hi * 128:(hi + 1) * 128, :])
        ksg = ksg + jnp.where(m, _gather_lane(ks_row, idx_lo), 0.0)
        nx = nx + jnp.where(m, _gather_lane(xyzT[0 + hi:1 + hi, :], idx_lo), 0.0)
        ny = ny + jnp.where(m, _gather_lane(xyzT[8 + hi:9 + hi, :], idx_lo), 0.0)
        nz = nz + jnp.where(m, _gather_lane(xyzT[16 + hi:17 + hi, :], idx_lo), 0.0)

    xyzb = xyz_ref[0]
    rx = xyzb[:, 0:1] - nx
    ry = xyzb[:, 1:2] - ny
    rz = xyzb[:, 2:3] - nz

    wp1 = wp1_ref[...]
    h = (rx[:, :, None] * wp1[0, :][None, None, :]
         + ry[:, :, None] * wp1[1, :][None, None, :]
         + rz[:, :, None] * wp1[2, :][None, None, :]
         + bp1_ref[0, :][None, None, :])
    r = jnp.maximum(h, 0.0)
    rw = jnp.sum(r * w2s_row[0, :][None, None, :], axis=2)

    logits = rw - ksg
    m = jnp.max(logits, axis=1, keepdims=True)
    e = jnp.exp(logits - m)
    attn = e / jnp.sum(e, axis=1, keepdims=True)

    gr = jnp.sum(attn[:, :, None] * r, axis=1)

    posb = pos_ref[0]
    g = jnp.zeros((BLK, D), _f32)
    for hi in range(NCHUNK):
        pc = posb[:, hi * 128:(hi + 1) * 128]
        ac = jnp.where(pc >= 0,
                       jnp.take_along_axis(attn, jnp.maximum(pc, 0), axis=1),
                       0.0)
        g = g + jnp.dot(ac, vfo[hi * 128:(hi + 1) * 128, :],
                        preferred_element_type=_f32)
    g = g + jnp.dot(gr, wp2wo_ref[...], preferred_element_type=_f32) + cvec
    out_ref[0] = featb_ref[0] + gamma_ref[...] * g


def _layer(feat, xyz, xyzT, idx, pos, wvwo, wp2wo, misc, wp1, bp1, gamma):
    return pl.pallas_call(
        _layer_body,
        grid=(BATCH, NBLK),
        in_specs=[
            pl.BlockSpec((1, N, D), lambda b, i: (b, 0, 0)),
            pl.BlockSpec((1, BLK, D), lambda b, i: (b, i, 0)),
            pl.BlockSpec((1, BLK, 3), lambda b, i: (b, i, 0)),
            pl.BlockSpec((1, 24, 128), lambda b, i: (b, 0, 0)),
            pl.BlockSpec((1, BLK, KNN), lambda b, i: (b, i, 0)),
            pl.BlockSpec((1, BLK, N), lambda b, i: (b, i, 0)),
            pl.BlockSpec((D, D), lambda b, i: (0, 0)),
            pl.BlockSpec((D, D), lambda b, i: (0, 0)),
            pl.BlockSpec((8, 128), lambda b, i: (0, 0)),
            pl.BlockSpec((3, 128), lambda b, i: (0, 0)),
            pl.BlockSpec((1, 128), lambda b, i: (0, 0)),
            pl.BlockSpec((1, 1), lambda b, i: (0, 0)),
        ],
        out_specs=pl.BlockSpec((1, BLK, D), lambda b, i: (b, i, 0)),
        out_shape=jax.ShapeDtypeStruct((BATCH, N, D), _f32),
    )(feat, feat, xyz, xyzT, idx, pos, wvwo, wp2wo, misc, wp1, bp1, gamma)


def kernel(xyz, feat, params):
    wvwo, wp2wo, misc = _prep(params)
    pos, xyzT = _knn(xyz)
    idx = _sc_compact(pos)
    for l in range(NLAYERS):
        p = params[l]
        feat = _layer(feat, xyz, xyzT, idx, pos,
                      wvwo[l], wp2wo[l], misc[l],
                      p['Wp1'], p['bp1'][None, :], p['gamma'][None, :])
    return feat


# speedup vs baseline: 14.4699x; 1.1584x over previous
import functools

import jax
import jax.numpy as jnp
from jax import lax
from jax.experimental import pallas as pl
from jax.experimental.pallas import tpu as pltpu
from jax.experimental.pallas import tpu_sc as plsc

BATCH = 2
N = 1024
D = 128
KNN = 128
NLAYERS = 4
BLK = 128
NBLK = N // BLK
NCHUNK = N // 128

_f32 = jnp.float32
_i32 = jnp.int32


def _dgT(a, b):
    return lax.dot_general(a, b, (((1,), (1,)), ((), ())),
                           preferred_element_type=_f32)


def _prep_body(wk_ref, wv_ref, wp2_ref, wo_ref, bv_ref, bp2_ref, bo_ref,
               wvwo_ref, wp2wo_ref, misc_ref):
    wo = wo_ref[0]
    wvwo_ref[0] = jnp.dot(wv_ref[0], wo, preferred_element_type=_f32)
    wp2wo_ref[0] = jnp.dot(wp2_ref[0], wo, preferred_element_type=_f32)
    ones = jnp.ones((1, D), _f32)
    wks = _dgT(ones, wk_ref[0])
    w2s = _dgT(ones, wp2_ref[0])
    cvec = lax.dot_general(bv_ref[0] + bp2_ref[0], wo,
                           (((1,), (0,)), ((), ())),
                           preferred_element_type=_f32) + bo_ref[0]
    misc_ref[0] = jnp.concatenate(
        [wks, w2s, cvec, jnp.zeros((5, 128), _f32)], axis=0)


def _prep(params):
    wk = jnp.stack([p['Wk'] for p in params])
    wv = jnp.stack([p['Wv'] for p in params])
    wp2 = jnp.stack([p['Wp2'] for p in params])
    wo = jnp.stack([p['Wo'] for p in params])
    bv = jnp.stack([p['bv'][None, :] for p in params])
    bp2 = jnp.stack([p['bp2'][None, :] for p in params])
    bo = jnp.stack([p['bo'][None, :] for p in params])
    full = pl.BlockSpec((1, D, D), lambda l: (l, 0, 0))
    row = pl.BlockSpec((1, 1, D), lambda l: (l, 0, 0))
    return pl.pallas_call(
        _prep_body,
        grid=(NLAYERS,),
        in_specs=[full, full, full, full, row, row, row],
        out_specs=[full, full, pl.BlockSpec((1, 8, 128), lambda l: (l, 0, 0))],
        out_shape=[
            jax.ShapeDtypeStruct((NLAYERS, D, D), _f32),
            jax.ShapeDtypeStruct((NLAYERS, D, D), _f32),
            jax.ShapeDtypeStruct((NLAYERS, 8, 128), _f32),
        ],
    )(wk, wv, wp2, wo, bv, bp2, bo)


def _sortable(d):
    i = lax.bitcast_convert_type(d, _i32)
    return jnp.where(i >= 0, i, i ^ jnp.int32(0x7FFFFFFF))


def _lane_cumsum(x):
    acc = x
    sh = 1
    while sh < N:
        z = jnp.zeros((BLK, sh), _f32)
        acc = acc + jnp.concatenate([z, acc[:, :-sh]], axis=1)
        sh *= 2
    return acc


def _knn_body(xyz_ref, xyzb_ref, pos_ref, xyzT_ref):
    i = pl.program_id(1)
    xyzf = xyz_ref[0]
    blk = xyzb_ref[0]
    s2 = jnp.sum(blk * blk, axis=1, keepdims=True)
    eye3 = jnp.eye(3, dtype=_f32)
    key_chunks = []
    xt_rows = []
    for hi in range(NCHUNK):
        ch = xyzf[hi * 128:(hi + 1) * 128, :]
        cross = _dgT(blk, ch)
        xt = _dgT(eye3, ch)
        d2 = jnp.sum(xt * xt, axis=0, keepdims=True)
        dist = s2 + d2 - 2.0 * cross
        key_chunks.append(_sortable(dist))
        xt_rows.append(xt)
    keys = jnp.concatenate(key_chunks, axis=1)

    @pl.when(i == 0)
    def _():
        xyzT_ref[0] = jnp.concatenate(
            [jnp.concatenate([xt_rows[hi][c:c + 1, :] for hi in range(NCHUNK)],
                             axis=0) for c in range(3)], axis=0)

    ones_row = jnp.ones((1, N), _f32)
    kf = jnp.float32(KNN)
    lo = jnp.full((BLK, 1), jnp.iinfo(jnp.int32).min, _i32)
    hi_b = jnp.full((BLK, 1), jnp.iinfo(jnp.int32).max, _i32)
    for _ in range(32):
        mid = (lo >> 1) + (hi_b >> 1) + (lo & hi_b & 1)
        ind = (keys <= mid).astype(_f32)
        cnt = _dgT(ind, ones_row)
        pred = cnt >= kf
        hi_b = jnp.where(pred, mid, hi_b)
        lo = jnp.where(pred, lo, mid + 1)
    thr = lo

    sel_lt = (keys < thr).astype(_f32)
    ties = (keys == thr).astype(_f32)
    cnt_lt = _dgT(sel_lt, ones_row)
    need = kf - cnt_lt
    tie_excl = _lane_cumsum(ties) - ties
    sel = sel_lt + ties * (tie_excl < need).astype(_f32)
    rank_excl = _lane_cumsum(sel) - sel
    pos_ref[0] = jnp.where(sel > 0.5, rank_excl.astype(_i32), jnp.int32(-1))


def _knn(xyz):
    return pl.pallas_call(
        _knn_body,
        grid=(BATCH, NBLK),
        in_specs=[pl.BlockSpec((1, N, 3), lambda b, i: (b, 0, 0)),
                  pl.BlockSpec((1, BLK, 3), lambda b, i: (b, i, 0))],
        out_specs=[
            pl.BlockSpec((1, BLK, N), lambda b, i: (b, i, 0)),
            pl.BlockSpec((1, 24, 128), lambda b, i: (b, 0, 0)),
        ],
        out_shape=[
            jax.ShapeDtypeStruct((BATCH, N, N), _i32),
            jax.ShapeDtypeStruct((BATCH, 24, 128), _f32),
        ],
    )(xyz, xyz)


_ROWS = BATCH * N
_NW = 32
_RPW = _ROWS // _NW
_RB = 8


def _sc_compact_body(pos_hbm, idx_hbm, pos_v, idx_v):
    wid = lax.axis_index("s") * 2 + lax.axis_index("c")

    def outer(t, _):
        base = wid * _RPW + t * _RB
        pltpu.sync_copy(pos_hbm.at[pl.ds(base * N, _RB * N)], pos_v)

        def chunk_body(ci, _):
            pv = pos_v[pl.ds(ci * 16, 16)]
            mask = pv >= 0
            pvc = jnp.maximum(pv, 0)
            r = (ci * 16) // N
            jv = lax.iota(_i32, 16) + (ci * 16 - r * N)
            plsc.store_scatter(idx_v, [pvc + r * KNN], jv, mask=mask)
            return 0

        lax.fori_loop(0, _RB * N // 16, chunk_body, 0)
        pltpu.sync_copy(idx_v, idx_hbm.at[pl.ds(base * KNN, _RB * KNN)])
        return 0

    lax.fori_loop(0, _RPW // _RB, outer, 0)


def _sc_compact(pos):
    mesh = plsc.VectorSubcoreMesh(core_axis_name="c", subcore_axis_name="s")
    kern = functools.partial(
        pl.kernel,
        mesh=mesh,
        compiler_params=pltpu.CompilerParams(needs_layout_passes=False),
        out_type=jax.ShapeDtypeStruct((_ROWS * KNN,), _i32),
        scratch_types=[
            pltpu.VMEM((_RB * N,), _i32),
            pltpu.VMEM((_RB * KNN,), _i32),
        ],
    )(_sc_compact_body)
    idx = kern(pos.reshape(_ROWS * N))
    return idx.reshape(BATCH, N, KNN)


def _gather_lane(table_row, idx_lo):
    t = jnp.broadcast_to(table_row, (BLK, 128))
    return jnp.take_along_axis(t, idx_lo, axis=1)


def _layer_body(feat_ref, featb_ref, xyz_ref, xyzT_ref, idx_ref, pos_ref,
                wvwo_ref, wp2wo_ref, misc_ref, wp1_ref, bp1_ref, gamma_ref,
                out_ref):
    featf = feat_ref[0]
    vfo = jnp.dot(featf, wvwo_ref[...], preferred_element_type=_f32)
    wks_row = misc_ref[0:1, :]
    w2s_row = misc_ref[1:2, :]
    cvec = misc_ref[2:3, :]

    idxb = idx_ref[0]
    idx_hi = idxb >> 7
    idx_lo = idxb & 127

    xyzT = xyzT_ref[0]
    ksg = jnp.zeros((BLK, 128), _f32)
    nx = jnp.zeros((BLK, 128), _f32)
    ny = jnp.zeros((BLK, 128), _f32)
    nz = jnp.zeros((BLK, 128), _f32)
    for hi in range(NCHUNK):
        m = (idx_hi == hi)
        ks_row = _dgT(wks_row, featf[hi * 128:(hi + 1) * 128, :])
        ksg = ksg + jnp.where(m, _gather_lane(ks_row, idx_lo), 0.0)
        nx = nx + jnp.where(m, _gather_lane(xyzT[0 + hi:1 + hi, :], idx_lo), 0.0)
        ny = ny + jnp.where(m, _gather_lane(xyzT[8 + hi:9 + hi, :], idx_lo), 0.0)
        nz = nz + jnp.where(m, _gather_lane(xyzT[16 + hi:17 + hi, :], idx_lo), 0.0)

    xyzb = xyz_ref[0]
    bf16 = jnp.bfloat16
    rx = (xyzb[:, 0:1] - nx).astype(bf16)
    ry = (xyzb[:, 1:2] - ny).astype(bf16)
    rz = (xyzb[:, 2:3] - nz).astype(bf16)

    wp1 = wp1_ref[...].astype(bf16)
    bp1b = bp1_ref[...].astype(bf16)
    h = (bp1b[0, :][None, None, :]
         + rz[:, :, None] * wp1[2, :][None, None, :])
    h = h + ry[:, :, None] * wp1[1, :][None, None, :]
    h = h + rx[:, :, None] * wp1[0, :][None, None, :]
    r = jnp.maximum(h, jnp.bfloat16(0.0))
    w2sb = w2s_row.astype(bf16)
    rw = jnp.sum(r * w2sb[0, :][None, None, :], axis=2,
                 dtype=_f32)

    logits = rw - ksg
    m = jnp.max(logits, axis=1, keepdims=True)
    e = jnp.exp(logits - m)
    attn = e / jnp.sum(e, axis=1, keepdims=True)

    gr = jnp.sum(attn.astype(bf16)[:, :, None] * r, axis=1,
                 dtype=_f32)

    posb = pos_ref[0]
    g = jnp.zeros((BLK, D), _f32)
    for hi in range(NCHUNK):
        pc = posb[:, hi * 128:(hi + 1) * 128]
        ac = jnp.where(pc >= 0,
                       jnp.take_along_axis(attn, jnp.maximum(pc, 0), axis=1),
                       0.0)
        g = g + jnp.dot(ac, vfo[hi * 128:(hi + 1) * 128, :],
                        preferred_element_type=_f32)
    g = g + jnp.dot(gr, wp2wo_ref[...], preferred_element_type=_f32) + cvec
    out_ref[0] = featb_ref[0] + gamma_ref[...] * g


def _layer(feat, xyz, xyzT, idx, pos, wvwo, wp2wo, misc, wp1, bp1, gamma):
    return pl.pallas_call(
        _layer_body,
        grid=(BATCH, NBLK),
        in_specs=[
            pl.BlockSpec((1, N, D), lambda b, i: (b, 0, 0)),
            pl.BlockSpec((1, BLK, D), lambda b, i: (b, i, 0)),
            pl.BlockSpec((1, BLK, 3), lambda b, i: (b, i, 0)),
            pl.BlockSpec((1, 24, 128), lambda b, i: (b, 0, 0)),
            pl.BlockSpec((1, BLK, KNN), lambda b, i: (b, i, 0)),
            pl.BlockSpec((1, BLK, N), lambda b, i: (b, i, 0)),
            pl.BlockSpec((D, D), lambda b, i: (0, 0)),
            pl.BlockSpec((D, D), lambda b, i: (0, 0)),
            pl.BlockSpec((8, 128), lambda b, i: (0, 0)),
            pl.BlockSpec((3, 128), lambda b, i: (0, 0)),
            pl.BlockSpec((1, 128), lambda b, i: (0, 0)),
            pl.BlockSpec((1, 1), lambda b, i: (0, 0)),
        ],
        out_specs=pl.BlockSpec((1, BLK, D), lambda b, i: (b, i, 0)),
        out_shape=jax.ShapeDtypeStruct((BATCH, N, D), _f32),
    )(feat, feat, xyz, xyzT, idx, pos, wvwo, wp2wo, misc, wp1, bp1, gamma)


def kernel(xyz, feat, params):
    wvwo, wp2wo, misc = _prep(params)
    pos, xyzT = _knn(xyz)
    idx = _sc_compact(pos)
    for l in range(NLAYERS):
        p = params[l]
        feat = _layer(feat, xyz, xyzT, idx, pos,
                      wvwo[l], wp2wo[l], misc[l],
                      p['Wp1'], p['bp1'][None, :], p['gamma'][None, :])
    return feat
